# baseline (speedup 1.0000x reference)
import functools

import jax
import jax.numpy as jnp
from jax import lax
from jax.experimental import pallas as pl
from jax.experimental.pallas import tpu as pltpu
from jax.experimental.pallas import tpu_sc as plsc

MAX_DEG = 512
EMBED = 64
B = 16
N = 4096
TOTAL_ROWS = B * N
NUM_WORKERS = 32
ROWS_PER_WORKER = TOTAL_ROWS // NUM_WORKERS
CHUNK = 128
NCHUNKS = ROWS_PER_WORKER // CHUNK
GROUPS = CHUNK // 16


def _body(din_hbm, dout_hbm, tin_hbm, tout_hbm, out_hbm,
          tin_v, tout_v, din_v, dout_v, obuf, sems):
    nc = 2
    wid = lax.axis_index("s") * nc + lax.axis_index("c")
    batch = wid // 2
    half = (wid % 2) * (N // 2)

    pltpu.sync_copy(tin_hbm, tin_v)
    pltpu.sync_copy(tout_hbm, tout_v)
    pltpu.sync_copy(din_hbm.at[batch, pl.ds(half, ROWS_PER_WORKER)], din_v)
    pltpu.sync_copy(dout_hbm.at[batch, pl.ds(half, ROWS_PER_WORKER)], dout_v)

    zeros16 = jnp.zeros((16,), jnp.int32)
    max16 = jnp.full((16,), MAX_DEG, jnp.int32)

    def compute_chunk(ci, rbase):
        @plsc.parallel_loop(0, GROUPS, unroll=1)
        def _group(g):
            off = ci * CHUNK + g * 16
            dv = jnp.minimum(jnp.maximum(din_v[pl.ds(off, 16)], zeros16), max16) * EMBED
            ev = jnp.minimum(jnp.maximum(dout_v[pl.ds(off, 16)], zeros16), max16) * EMBED
            for j in range(16):
                db = dv[j]
                eb = ev[j]
                rb = g * 16 + j
                vis = [tin_v[pl.ds(db + k, 16)] for k in range(0, EMBED, 16)]
                vos = [tout_v[pl.ds(eb + k, 16)] for k in range(0, EMBED, 16)]
                for i, k in enumerate(range(0, EMBED, 16)):
                    obuf[rbase + rb, pl.ds(k, 16)] = vis[i] + vos[i]

    def out_slice(ci):
        return out_hbm.at[pl.ds(batch * N + half + ci * CHUNK, CHUNK), :]

    @pl.loop(0, NCHUNKS)
    def _chunk(ci):
        b = ci % 2
        rbase = b * CHUNK

        @pl.when(ci >= 2)
        def _():
            pltpu.make_async_copy(obuf.at[pl.ds(rbase, CHUNK), :],
                                  out_slice(ci - 2), sems.at[b]).wait()

        compute_chunk(ci, rbase)
        pltpu.make_async_copy(obuf.at[pl.ds(rbase, CHUNK), :],
                              out_slice(ci), sems.at[b]).start()

    for b in range(2):
        pltpu.make_async_copy(obuf.at[pl.ds(b * CHUNK, CHUNK), :],
                              out_slice(NCHUNKS - 2 + b), sems.at[b]).wait()


def kernel(in_degree, out_degree, table_in, table_out):
    mesh = plsc.VectorSubcoreMesh(core_axis_name="c", subcore_axis_name="s")
    run = functools.partial(
        pl.kernel,
        mesh=mesh,
        compiler_params=pltpu.CompilerParams(
            needs_layout_passes=False, use_tc_tiling_on_sc=True),
        out_type=jax.ShapeDtypeStruct((TOTAL_ROWS, EMBED), jnp.float32),
        scratch_types=[
            pltpu.VMEM(((MAX_DEG + 1) * EMBED,), jnp.float32),
            pltpu.VMEM(((MAX_DEG + 1) * EMBED,), jnp.float32),
            pltpu.VMEM((ROWS_PER_WORKER,), jnp.int32),
            pltpu.VMEM((ROWS_PER_WORKER,), jnp.int32),
            pltpu.VMEM((2 * CHUNK, EMBED), jnp.float32),
            pltpu.SemaphoreType.DMA((2,)),
        ],
    )(_body)
    out = run(in_degree.astype(jnp.int32), out_degree.astype(jnp.int32),
              table_in.reshape(-1), table_out.reshape(-1))
    return out.reshape(B, N, EMBED)


# speedup vs baseline: 1.2878x; 1.2878x over previous
import functools

import jax
import jax.numpy as jnp
from jax import lax
from jax.experimental import pallas as pl
from jax.experimental.pallas import tpu as pltpu
from jax.experimental.pallas import tpu_sc as plsc

MAX_DEG = 512
EMBED = 64
B = 16
N = 4096
TOTAL_ROWS = B * N
NUM_WORKERS = 32
ROWS_PER_WORKER = TOTAL_ROWS // NUM_WORKERS
CHUNK = 128
NCHUNKS = ROWS_PER_WORKER // CHUNK
GROUPS = CHUNK // 16


def _body(din_hbm, dout_hbm, tin_hbm, tout_hbm, out_hbm,
          tin_v, tout_v, din_v, dout_v, obuf, sems):
    nc = 2
    wid = lax.axis_index("s") * nc + lax.axis_index("c")
    batch = wid // 2
    half = (wid % 2) * (N // 2)

    pltpu.sync_copy(tin_hbm, tin_v)
    pltpu.sync_copy(tout_hbm, tout_v)
    pltpu.sync_copy(din_hbm.at[batch, pl.ds(half, ROWS_PER_WORKER)], din_v)
    pltpu.sync_copy(dout_hbm.at[batch, pl.ds(half, ROWS_PER_WORKER)], dout_v)

    zeros16 = jnp.zeros((16,), jnp.int32)
    max16 = jnp.full((16,), MAX_DEG, jnp.int32)
    step16 = jnp.full((16,), MAX_DEG + 1, jnp.int32)

    def compute_chunk(ci, cbase):
        @plsc.parallel_loop(0, GROUPS, unroll=1)
        def _group(g):
            off = ci * CHUNK + g * 16
            ii = jnp.minimum(jnp.maximum(din_v[pl.ds(off, 16)], zeros16), max16)
            oo = jnp.minimum(jnp.maximum(dout_v[pl.ds(off, 16)], zeros16), max16)
            col = cbase + g * 16
            for e in range(EMBED):
                vi = plsc.load_gather(tin_v, [ii + e * (MAX_DEG + 1)])
                vo = plsc.load_gather(tout_v, [oo + e * (MAX_DEG + 1)])
                obuf[e, pl.ds(col, 16)] = vi + vo

    def out_slice(ci):
        return out_hbm.at[batch, :, pl.ds(half + ci * CHUNK, CHUNK)]

    @pl.loop(0, NCHUNKS)
    def _chunk(ci):
        b = ci % 2
        cbase = b * CHUNK

        @pl.when(ci >= 2)
        def _():
            pltpu.make_async_copy(obuf.at[:, pl.ds(cbase, CHUNK)],
                                  out_slice(ci - 2), sems.at[b]).wait()

        compute_chunk(ci, cbase)
        pltpu.make_async_copy(obuf.at[:, pl.ds(cbase, CHUNK)],
                              out_slice(ci), sems.at[b]).start()

    for b in range(2):
        pltpu.make_async_copy(obuf.at[:, pl.ds(b * CHUNK, CHUNK)],
                              out_slice(NCHUNKS - 2 + b), sems.at[b]).wait()


def kernel(in_degree, out_degree, table_in, table_out):
    mesh = plsc.VectorSubcoreMesh(core_axis_name="c", subcore_axis_name="s")
    run = functools.partial(
        pl.kernel,
        mesh=mesh,
        compiler_params=pltpu.CompilerParams(needs_layout_passes=False),
        out_type=jax.ShapeDtypeStruct((B, EMBED, N), jnp.float32),
        scratch_types=[
            pltpu.VMEM((EMBED * (MAX_DEG + 1),), jnp.float32),
            pltpu.VMEM((EMBED * (MAX_DEG + 1),), jnp.float32),
            pltpu.VMEM((ROWS_PER_WORKER,), jnp.int32),
            pltpu.VMEM((ROWS_PER_WORKER,), jnp.int32),
            pltpu.VMEM((EMBED, 2 * CHUNK), jnp.float32),
            pltpu.SemaphoreType.DMA((2,)),
        ],
    )(_body)
    out = run(in_degree.astype(jnp.int32), out_degree.astype(jnp.int32),
              table_in.T.reshape(-1), table_out.T.reshape(-1))
    return out.transpose(0, 2, 1)
